# baseline (speedup 1.0000x reference)
import functools

import jax
import jax.numpy as jnp
from jax import lax
from jax.experimental import pallas as pl
from jax.experimental.pallas import tpu as pltpu
from jax.experimental.pallas import tpu_sc as plsc

BATCH = 16384
DIM = 64
NUM_ROWS = 1000000
NUM_MEMS = 20
MARGIN = 1.0

_NC = 2
_NS = 16
_NW = _NC * _NS
_ROWS_PER_W = BATCH // _NW

_TBLK = 2048


def _transpose_body(tt_ref, out_ref):
    out_ref[...] = jnp.swapaxes(tt_ref[...], 0, 1)


def _tc_convert(tt):
    grid = pl.cdiv(NUM_ROWS, _TBLK)
    return pl.pallas_call(
        _transpose_body,
        grid=(grid,),
        in_specs=[pl.BlockSpec((DIM, _TBLK), lambda i: (0, i))],
        out_specs=pl.BlockSpec((_TBLK, DIM), lambda i: (i, 0)),
        out_shape=jax.ShapeDtypeStruct((NUM_ROWS, DIM), jnp.float32),
    )(tt)


def _gather_item_body(pid, nid, iemb, pe_out, ne_out,
                      pidx_v, nidx_v, pe_v, ne_v, s1, s2):
    wid = lax.axis_index("s") * _NC + lax.axis_index("c")
    base = wid * _ROWS_PER_W
    sl = pl.ds(base, _ROWS_PER_W)
    pltpu.sync_copy(pid.at[sl], pidx_v)
    pltpu.sync_copy(nid.at[sl], nidx_v)
    cp = pltpu.async_copy(iemb.at[pidx_v], pe_v, s1)
    cn = pltpu.async_copy(iemb.at[nidx_v], ne_v, s2)
    cp.wait()
    pltpu.sync_copy(pe_v, pe_out.at[sl])
    cn.wait()
    pltpu.sync_copy(ne_v, ne_out.at[sl])


def _gather_user_body(uid, uemb, ue_out, uidx_v, ue_v, s0):
    wid = lax.axis_index("s") * _NC + lax.axis_index("c")
    base = wid * _ROWS_PER_W
    sl = pl.ds(base, _ROWS_PER_W)
    pltpu.sync_copy(uid.at[sl], uidx_v)
    pltpu.async_copy(uemb.at[uidx_v], ue_v, s0).wait()
    pltpu.sync_copy(ue_v, ue_out.at[sl])


def _sc_gather_item(pid, nid, iemb):
    mesh = plsc.VectorSubcoreMesh(core_axis_name="c", subcore_axis_name="s")
    f = pl.kernel(
        _gather_item_body,
        out_type=[jax.ShapeDtypeStruct((BATCH, DIM), jnp.float32)] * 2,
        mesh=mesh,
        scratch_types=[
            pltpu.VMEM((_ROWS_PER_W,), jnp.int32),
            pltpu.VMEM((_ROWS_PER_W,), jnp.int32),
            pltpu.VMEM((_ROWS_PER_W, DIM), jnp.float32),
            pltpu.VMEM((_ROWS_PER_W, DIM), jnp.float32),
            pltpu.SemaphoreType.DMA,
            pltpu.SemaphoreType.DMA,
        ],
        compiler_params=pltpu.CompilerParams(use_tc_tiling_on_sc=False),
    )
    return f(pid, nid, iemb)


def _sc_gather_user(uid, uemb):
    mesh = plsc.VectorSubcoreMesh(core_axis_name="c", subcore_axis_name="s")
    f = pl.kernel(
        _gather_user_body,
        out_type=jax.ShapeDtypeStruct((BATCH, DIM), jnp.float32),
        mesh=mesh,
        scratch_types=[
            pltpu.VMEM((_ROWS_PER_W,), jnp.int32),
            pltpu.VMEM((_ROWS_PER_W, DIM), jnp.float32),
            pltpu.SemaphoreType.DMA,
        ],
        compiler_params=pltpu.CompilerParams(use_tc_tiling_on_sc=False),
    )
    return f(uid, uemb)


_BLK = 2048


def _compute_body(key_ref, mem_ref, ue_ref, pe_ref, ne_ref, out_ref):
    ue = ue_ref[...]
    pe = pe_ref[...]
    ne = ne_ref[...]
    s = ue * pe
    logits = jnp.dot(s, key_ref[...], preferred_element_type=jnp.float32)
    m = jnp.max(logits, axis=-1, keepdims=True)
    w = jnp.exp(logits - m)
    attn = w / jnp.sum(w, axis=-1, keepdims=True)
    lat = jnp.dot(attn, mem_ref[...], preferred_element_type=jnp.float32)
    diff = ue + lat
    pos_d = jnp.sum(jnp.square(diff - pe), axis=-1)
    neg_d = jnp.sum(jnp.square(diff - ne), axis=-1)
    blk = jnp.sum(jnp.maximum(MARGIN + pos_d - neg_d, 0.0))

    @pl.when(pl.program_id(0) == 0)
    def _():
        out_ref[0, 0] = 0.0

    out_ref[0, 0] += blk


def _tc_compute(ue, pe, ne, user_item_key, memories):
    grid = BATCH // _BLK
    emb_spec = pl.BlockSpec((_BLK, DIM), lambda i: (i, 0))
    out = pl.pallas_call(
        _compute_body,
        grid=(grid,),
        in_specs=[
            pl.BlockSpec((DIM, NUM_MEMS), lambda i: (0, 0)),
            pl.BlockSpec((NUM_MEMS, DIM), lambda i: (0, 0)),
            emb_spec, emb_spec, emb_spec,
        ],
        out_specs=pl.BlockSpec(memory_space=pltpu.SMEM),
        out_shape=jax.ShapeDtypeStruct((1, 1), jnp.float32),
    )(user_item_key, memories, ue, pe, ne)
    return out[0, 0]


def kernel(user_ids, pos_ids, neg_ids, user_emb, item_emb, user_item_key, memories):
    uid = user_ids.astype(jnp.int32)
    pid = pos_ids.astype(jnp.int32)
    nid = neg_ids.astype(jnp.int32)
    iemb_cvt = _tc_convert(jnp.swapaxes(item_emb, 0, 1))
    pe, ne = _sc_gather_item(pid, nid, iemb_cvt)
    uemb_cvt = _tc_convert(jnp.swapaxes(user_emb, 0, 1))
    ue = _sc_gather_user(uid, uemb_cvt)
    return _tc_compute(ue, pe, ne, user_item_key, memories)


# speedup vs baseline: 1.3323x; 1.3323x over previous
import functools

import jax
import jax.numpy as jnp
from jax import lax
from jax.experimental import pallas as pl
from jax.experimental.pallas import tpu as pltpu
from jax.experimental.pallas import tpu_sc as plsc

BATCH = 16384
DIM = 64
NUM_ROWS = 1000000
NUM_MEMS = 20
MARGIN = 1.0

_NC = 2
_NS = 16
_NW = _NC * _NS
_ROWS_PER_W = BATCH // _NW

_TBLK = 16384


def _transpose_body(tt_ref, out_ref):
    out_ref[...] = jnp.swapaxes(tt_ref[...], 0, 1)


def _tc_convert(tt):
    grid = pl.cdiv(NUM_ROWS, _TBLK)
    return pl.pallas_call(
        _transpose_body,
        grid=(grid,),
        in_specs=[pl.BlockSpec((DIM, _TBLK), lambda i: (0, i))],
        out_specs=pl.BlockSpec((_TBLK, DIM), lambda i: (i, 0)),
        out_shape=jax.ShapeDtypeStruct((NUM_ROWS, DIM), jnp.float32),
    )(tt)


def _gather_item_body(pid, nid, iemb, pe_out, ne_out,
                      pidx_v, nidx_v, pe_v, ne_v, s1, s2):
    wid = lax.axis_index("s") * _NC + lax.axis_index("c")
    base = wid * _ROWS_PER_W
    sl = pl.ds(base, _ROWS_PER_W)
    pltpu.sync_copy(pid.at[sl], pidx_v)
    pltpu.sync_copy(nid.at[sl], nidx_v)
    cp = pltpu.async_copy(iemb.at[pidx_v], pe_v, s1)
    cn = pltpu.async_copy(iemb.at[nidx_v], ne_v, s2)
    cp.wait()
    pltpu.sync_copy(pe_v, pe_out.at[sl])
    cn.wait()
    pltpu.sync_copy(ne_v, ne_out.at[sl])


def _gather_user_body(uid, uemb, ue_out, uidx_v, ue_v, s0):
    wid = lax.axis_index("s") * _NC + lax.axis_index("c")
    base = wid * _ROWS_PER_W
    sl = pl.ds(base, _ROWS_PER_W)
    pltpu.sync_copy(uid.at[sl], uidx_v)
    pltpu.async_copy(uemb.at[uidx_v], ue_v, s0).wait()
    pltpu.sync_copy(ue_v, ue_out.at[sl])


def _sc_gather_item(pid, nid, iemb):
    mesh = plsc.VectorSubcoreMesh(core_axis_name="c", subcore_axis_name="s")
    f = pl.kernel(
        _gather_item_body,
        out_type=[jax.ShapeDtypeStruct((BATCH, DIM), jnp.float32)] * 2,
        mesh=mesh,
        scratch_types=[
            pltpu.VMEM((_ROWS_PER_W,), jnp.int32),
            pltpu.VMEM((_ROWS_PER_W,), jnp.int32),
            pltpu.VMEM((_ROWS_PER_W, DIM), jnp.float32),
            pltpu.VMEM((_ROWS_PER_W, DIM), jnp.float32),
            pltpu.SemaphoreType.DMA,
            pltpu.SemaphoreType.DMA,
        ],
        compiler_params=pltpu.CompilerParams(use_tc_tiling_on_sc=False),
    )
    return f(pid, nid, iemb)


def _sc_gather_user(uid, uemb):
    mesh = plsc.VectorSubcoreMesh(core_axis_name="c", subcore_axis_name="s")
    f = pl.kernel(
        _gather_user_body,
        out_type=jax.ShapeDtypeStruct((BATCH, DIM), jnp.float32),
        mesh=mesh,
        scratch_types=[
            pltpu.VMEM((_ROWS_PER_W,), jnp.int32),
            pltpu.VMEM((_ROWS_PER_W, DIM), jnp.float32),
            pltpu.SemaphoreType.DMA,
        ],
        compiler_params=pltpu.CompilerParams(use_tc_tiling_on_sc=False),
    )
    return f(uid, uemb)


_BLK = 2048


def _compute_body(key_ref, mem_ref, ue_ref, pe_ref, ne_ref, out_ref):
    ue = ue_ref[...]
    pe = pe_ref[...]
    ne = ne_ref[...]
    s = ue * pe
    logits = jnp.dot(s, key_ref[...], preferred_element_type=jnp.float32)
    m = jnp.max(logits, axis=-1, keepdims=True)
    w = jnp.exp(logits - m)
    attn = w / jnp.sum(w, axis=-1, keepdims=True)
    lat = jnp.dot(attn, mem_ref[...], preferred_element_type=jnp.float32)
    diff = ue + lat
    pos_d = jnp.sum(jnp.square(diff - pe), axis=-1)
    neg_d = jnp.sum(jnp.square(diff - ne), axis=-1)
    blk = jnp.sum(jnp.maximum(MARGIN + pos_d - neg_d, 0.0))

    @pl.when(pl.program_id(0) == 0)
    def _():
        out_ref[0, 0] = 0.0

    out_ref[0, 0] += blk


def _tc_compute(ue, pe, ne, user_item_key, memories):
    grid = BATCH // _BLK
    emb_spec = pl.BlockSpec((_BLK, DIM), lambda i: (i, 0))
    out = pl.pallas_call(
        _compute_body,
        grid=(grid,),
        in_specs=[
            pl.BlockSpec((DIM, NUM_MEMS), lambda i: (0, 0)),
            pl.BlockSpec((NUM_MEMS, DIM), lambda i: (0, 0)),
            emb_spec, emb_spec, emb_spec,
        ],
        out_specs=pl.BlockSpec(memory_space=pltpu.SMEM),
        out_shape=jax.ShapeDtypeStruct((1, 1), jnp.float32),
    )(user_item_key, memories, ue, pe, ne)
    return out[0, 0]


def kernel(user_ids, pos_ids, neg_ids, user_emb, item_emb, user_item_key, memories):
    uid = user_ids.astype(jnp.int32)
    pid = pos_ids.astype(jnp.int32)
    nid = neg_ids.astype(jnp.int32)
    iemb_cvt = _tc_convert(jnp.swapaxes(item_emb, 0, 1))
    pe, ne = _sc_gather_item(pid, nid, iemb_cvt)
    uemb_cvt = _tc_convert(jnp.swapaxes(user_emb, 0, 1))
    ue = _sc_gather_user(uid, uemb_cvt)
    return _tc_compute(ue, pe, ne, user_item_key, memories)


# speedup vs baseline: 1.3469x; 1.0110x over previous
import functools

import jax
import jax.numpy as jnp
from jax import lax
from jax.experimental import pallas as pl
from jax.experimental.pallas import tpu as pltpu
from jax.experimental.pallas import tpu_sc as plsc

BATCH = 16384
DIM = 64
NUM_ROWS = 1000000
NUM_MEMS = 20
MARGIN = 1.0

_NC = 2
_NS = 16
_NW = _NC * _NS
_ROWS_PER_W = BATCH // _NW

_TBLK = 32768


def _transpose_body(eye_ref, tt_ref, out_ref):
    out_ref[...] = lax.dot_general(
        tt_ref[...], eye_ref[...], (((0,), (0,)), ((), ())),
        preferred_element_type=jnp.float32)


def _tc_convert(tt, eye):
    grid = pl.cdiv(NUM_ROWS, _TBLK)
    return pl.pallas_call(
        _transpose_body,
        grid=(grid,),
        in_specs=[
            pl.BlockSpec((DIM, DIM), lambda i: (0, 0)),
            pl.BlockSpec((DIM, _TBLK), lambda i: (0, i)),
        ],
        out_specs=pl.BlockSpec((_TBLK, DIM), lambda i: (i, 0)),
        out_shape=jax.ShapeDtypeStruct((NUM_ROWS, DIM), jnp.float32),
        compiler_params=pltpu.CompilerParams(vmem_limit_bytes=100 * 1024 * 1024),
    )(eye, tt)


def _gather_item_body(pid, nid, iemb, pe_out, ne_out,
                      pidx_v, nidx_v, pe_v, ne_v, s1, s2):
    wid = lax.axis_index("s") * _NC + lax.axis_index("c")
    base = wid * _ROWS_PER_W
    sl = pl.ds(base, _ROWS_PER_W)
    pltpu.sync_copy(pid.at[sl], pidx_v)
    pltpu.sync_copy(nid.at[sl], nidx_v)
    cp = pltpu.async_copy(iemb.at[pidx_v], pe_v, s1)
    cn = pltpu.async_copy(iemb.at[nidx_v], ne_v, s2)
    cp.wait()
    pltpu.sync_copy(pe_v, pe_out.at[sl])
    cn.wait()
    pltpu.sync_copy(ne_v, ne_out.at[sl])


def _gather_user_body(uid, uemb, ue_out, uidx_v, ue_v, s0):
    wid = lax.axis_index("s") * _NC + lax.axis_index("c")
    base = wid * _ROWS_PER_W
    sl = pl.ds(base, _ROWS_PER_W)
    pltpu.sync_copy(uid.at[sl], uidx_v)
    pltpu.async_copy(uemb.at[uidx_v], ue_v, s0).wait()
    pltpu.sync_copy(ue_v, ue_out.at[sl])


def _sc_gather_item(pid, nid, iemb):
    mesh = plsc.VectorSubcoreMesh(core_axis_name="c", subcore_axis_name="s")
    f = pl.kernel(
        _gather_item_body,
        out_type=[jax.ShapeDtypeStruct((BATCH, DIM), jnp.float32)] * 2,
        mesh=mesh,
        scratch_types=[
            pltpu.VMEM((_ROWS_PER_W,), jnp.int32),
            pltpu.VMEM((_ROWS_PER_W,), jnp.int32),
            pltpu.VMEM((_ROWS_PER_W, DIM), jnp.float32),
            pltpu.VMEM((_ROWS_PER_W, DIM), jnp.float32),
            pltpu.SemaphoreType.DMA,
            pltpu.SemaphoreType.DMA,
        ],
        compiler_params=pltpu.CompilerParams(use_tc_tiling_on_sc=False),
    )
    return f(pid, nid, iemb)


def _sc_gather_user(uid, uemb):
    mesh = plsc.VectorSubcoreMesh(core_axis_name="c", subcore_axis_name="s")
    f = pl.kernel(
        _gather_user_body,
        out_type=jax.ShapeDtypeStruct((BATCH, DIM), jnp.float32),
        mesh=mesh,
        scratch_types=[
            pltpu.VMEM((_ROWS_PER_W,), jnp.int32),
            pltpu.VMEM((_ROWS_PER_W, DIM), jnp.float32),
            pltpu.SemaphoreType.DMA,
        ],
        compiler_params=pltpu.CompilerParams(use_tc_tiling_on_sc=False),
    )
    return f(uid, uemb)


_BLK = 2048


def _compute_body(key_ref, mem_ref, ue_ref, pe_ref, ne_ref, out_ref):
    ue = ue_ref[...]
    pe = pe_ref[...]
    ne = ne_ref[...]
    s = ue * pe
    logits = jnp.dot(s, key_ref[...], preferred_element_type=jnp.float32)
    m = jnp.max(logits, axis=-1, keepdims=True)
    w = jnp.exp(logits - m)
    attn = w / jnp.sum(w, axis=-1, keepdims=True)
    lat = jnp.dot(attn, mem_ref[...], preferred_element_type=jnp.float32)
    diff = ue + lat
    pos_d = jnp.sum(jnp.square(diff - pe), axis=-1)
    neg_d = jnp.sum(jnp.square(diff - ne), axis=-1)
    blk = jnp.sum(jnp.maximum(MARGIN + pos_d - neg_d, 0.0))

    @pl.when(pl.program_id(0) == 0)
    def _():
        out_ref[0, 0] = 0.0

    out_ref[0, 0] += blk


def _tc_compute(ue, pe, ne, user_item_key, memories):
    grid = BATCH // _BLK
    emb_spec = pl.BlockSpec((_BLK, DIM), lambda i: (i, 0))
    out = pl.pallas_call(
        _compute_body,
        grid=(grid,),
        in_specs=[
            pl.BlockSpec((DIM, NUM_MEMS), lambda i: (0, 0)),
            pl.BlockSpec((NUM_MEMS, DIM), lambda i: (0, 0)),
            emb_spec, emb_spec, emb_spec,
        ],
        out_specs=pl.BlockSpec(memory_space=pltpu.SMEM),
        out_shape=jax.ShapeDtypeStruct((1, 1), jnp.float32),
    )(user_item_key, memories, ue, pe, ne)
    return out[0, 0]


def kernel(user_ids, pos_ids, neg_ids, user_emb, item_emb, user_item_key, memories):
    uid = user_ids.astype(jnp.int32)
    pid = pos_ids.astype(jnp.int32)
    nid = neg_ids.astype(jnp.int32)
    eye = jnp.eye(DIM, dtype=jnp.float32)
    iemb_cvt = _tc_convert(jnp.swapaxes(item_emb, 0, 1), eye)
    pe, ne = _sc_gather_item(pid, nid, iemb_cvt)
    uemb_cvt = _tc_convert(jnp.swapaxes(user_emb, 0, 1), eye)
    ue = _sc_gather_user(uid, uemb_cvt)
    return _tc_compute(ue, pe, ne, user_item_key, memories)


# speedup vs baseline: 1.5101x; 1.1212x over previous
import functools

import jax
import jax.numpy as jnp
from jax import lax
from jax.experimental import pallas as pl
from jax.experimental.pallas import tpu as pltpu
from jax.experimental.pallas import tpu_sc as plsc

BATCH = 16384
DIM = 64
PAIR = 2 * DIM
NUM_ROWS = 1000000
NUM_MEMS = 20
MARGIN = 1.0

_NC = 2
_NS = 16
_NW = _NC * _NS
_ROWS_PER_W = BATCH // _NW
_CHUNK = _ROWS_PER_W // 2


def _gather_body(uid2, pid2, nid2, uemb2, iemb2, ue_out, pe_out, ne_out,
                 uidx_v, pidx_v, nidx_v, ue_v, pe_v, ne_v, s0, s1, s2):
    wid = lax.axis_index("s") * _NC + lax.axis_index("c")
    base = wid * _ROWS_PER_W
    for half in range(2):
        sl = pl.ds(base + half * _CHUNK, _CHUNK)
        pltpu.sync_copy(uid2.at[sl], uidx_v)
        pltpu.sync_copy(pid2.at[sl], pidx_v)
        pltpu.sync_copy(nid2.at[sl], nidx_v)
        cu = pltpu.async_copy(uemb2.at[uidx_v], ue_v, s0)
        cp = pltpu.async_copy(iemb2.at[pidx_v], pe_v, s1)
        cn = pltpu.async_copy(iemb2.at[nidx_v], ne_v, s2)
        cu.wait()
        pltpu.sync_copy(ue_v, ue_out.at[sl])
        cp.wait()
        pltpu.sync_copy(pe_v, pe_out.at[sl])
        cn.wait()
        pltpu.sync_copy(ne_v, ne_out.at[sl])


def _sc_gather(uid2, pid2, nid2, uemb2, iemb2):
    mesh = plsc.VectorSubcoreMesh(core_axis_name="c", subcore_axis_name="s")
    f = pl.kernel(
        _gather_body,
        out_type=[jax.ShapeDtypeStruct((BATCH, PAIR), jnp.float32)] * 3,
        mesh=mesh,
        scratch_types=[
            pltpu.VMEM((_CHUNK,), jnp.int32),
            pltpu.VMEM((_CHUNK,), jnp.int32),
            pltpu.VMEM((_CHUNK,), jnp.int32),
            pltpu.VMEM((_CHUNK, PAIR), jnp.float32),
            pltpu.VMEM((_CHUNK, PAIR), jnp.float32),
            pltpu.VMEM((_CHUNK, PAIR), jnp.float32),
            pltpu.SemaphoreType.DMA,
            pltpu.SemaphoreType.DMA,
            pltpu.SemaphoreType.DMA,
        ],
    )
    return f(uid2, pid2, nid2, uemb2, iemb2)


_BLK = 2048


def _compute_body(key_ref, mem_ref, uh_ref, ph_ref, nh_ref,
                  ue_ref, pe_ref, ne_ref, out_ref):
    uh = uh_ref[...]
    ph = ph_ref[...]
    nh = nh_ref[...]
    ue = ue_ref[:, :DIM] * (1.0 - uh) + ue_ref[:, DIM:] * uh
    pe = pe_ref[:, :DIM] * (1.0 - ph) + pe_ref[:, DIM:] * ph
    ne = ne_ref[:, :DIM] * (1.0 - nh) + ne_ref[:, DIM:] * nh
    s = ue * pe
    logits = jnp.dot(s, key_ref[...], preferred_element_type=jnp.float32)
    m = jnp.max(logits, axis=-1, keepdims=True)
    w = jnp.exp(logits - m)
    attn = w / jnp.sum(w, axis=-1, keepdims=True)
    lat = jnp.dot(attn, mem_ref[...], preferred_element_type=jnp.float32)
    diff = ue + lat
    pos_d = jnp.sum(jnp.square(diff - pe), axis=-1)
    neg_d = jnp.sum(jnp.square(diff - ne), axis=-1)
    blk = jnp.sum(jnp.maximum(MARGIN + pos_d - neg_d, 0.0))

    @pl.when(pl.program_id(0) == 0)
    def _():
        out_ref[0, 0] = 0.0

    out_ref[0, 0] += blk


def _tc_compute(ue, pe, ne, uh, ph, nh, user_item_key, memories):
    grid = BATCH // _BLK
    emb_spec = pl.BlockSpec((_BLK, PAIR), lambda i: (i, 0))
    h_spec = pl.BlockSpec((_BLK, 1), lambda i: (i, 0))
    out = pl.pallas_call(
        _compute_body,
        grid=(grid,),
        in_specs=[
            pl.BlockSpec((DIM, NUM_MEMS), lambda i: (0, 0)),
            pl.BlockSpec((NUM_MEMS, DIM), lambda i: (0, 0)),
            h_spec, h_spec, h_spec,
            emb_spec, emb_spec, emb_spec,
        ],
        out_specs=pl.BlockSpec(memory_space=pltpu.SMEM),
        out_shape=jax.ShapeDtypeStruct((1, 1), jnp.float32),
    )(user_item_key, memories, uh, ph, nh, ue, pe, ne)
    return out[0, 0]


def kernel(user_ids, pos_ids, neg_ids, user_emb, item_emb, user_item_key, memories):
    uid = user_ids.astype(jnp.int32)
    pid = pos_ids.astype(jnp.int32)
    nid = neg_ids.astype(jnp.int32)
    uemb2 = jnp.reshape(user_emb, (NUM_ROWS // 2, PAIR))
    iemb2 = jnp.reshape(item_emb, (NUM_ROWS // 2, PAIR))
    ue, pe, ne = _sc_gather(uid // 2, pid // 2, nid // 2, uemb2, iemb2)
    uh = (uid % 2).astype(jnp.float32).reshape(BATCH, 1)
    ph = (pid % 2).astype(jnp.float32).reshape(BATCH, 1)
    nh = (nid % 2).astype(jnp.float32).reshape(BATCH, 1)
    return _tc_compute(ue, pe, ne, uh, ph, nh, user_item_key, memories)
